# baseline (speedup 1.0000x reference)
import functools

import jax
import jax.numpy as jnp
from jax import lax
from jax.experimental import pallas as pl
from jax.experimental.pallas import tpu as pltpu
from jax.experimental.pallas import tpu_sc as plsc

NC = 2
NS = 16
NW = NC * NS
LANES = 128
PITCH = 16


@functools.partial(jax.jit, static_argnums=(2,))
def _sc_gather(abs_idx, table_flat, d):
    n_chunks = abs_idx.shape[1]
    per_w = n_chunks * LANES
    flat_w = per_w * PITCH
    mesh = plsc.VectorSubcoreMesh(core_axis_name="c", subcore_axis_name="s")

    @functools.partial(
        pl.kernel,
        out_type=jax.ShapeDtypeStruct((NW * flat_w,), jnp.float32),
        mesh=mesh,
        scratch_types=[
            pltpu.VMEM((n_chunks, d, LANES), jnp.int32),
            pltpu.VMEM((d, per_w), jnp.float32),
            pltpu.VMEM((flat_w,), jnp.float32),
            pltpu.SemaphoreType.DMA,
            pltpu.SemaphoreType.DMA,
        ],
        compiler_params=pltpu.CompilerParams(needs_layout_passes=False),
    )
    def gather_kernel(idx_hbm, table_hbm, out_hbm, idx_v, feat_v, g_v, sem, sem2):
        wid = lax.axis_index("s") * NC + lax.axis_index("c")
        pltpu.sync_copy(idx_hbm.at[wid], idx_v)

        zeros16 = jnp.zeros((16,), jnp.float32)

        def zero_body(z, _):
            g_v[pl.ds(z * 16, 16)] = zeros16
            return 0

        lax.fori_loop(0, flat_w // 16, zero_body, 0)

        def chunk_body(c, _):
            copies = []
            for f in range(d):
                copies.append(
                    pltpu.async_copy(
                        table_hbm.at[idx_v.at[c, f]],
                        feat_v.at[f, pl.ds(c * LANES, LANES)],
                        sem,
                    )
                )
            for cp in copies:
                cp.wait()
            return 0

        lax.fori_loop(0, n_chunks, chunk_body, 0)

        lane16 = lax.iota(jnp.int32, 16) * PITCH

        def asm_f(f, _):
            def asm_g(g, _):
                v = feat_v[f, pl.ds(g * 16, 16)]
                plsc.store_scatter(g_v, [lane16 + (g * (16 * PITCH) + f)], v)
                return 0

            lax.fori_loop(0, per_w // 16, asm_g, 0)
            return 0

        lax.fori_loop(0, d, asm_f, 0)

        pltpu.async_copy(g_v, out_hbm.at[pl.ds(wid * flat_w, flat_w)], sem2).wait()

    return gather_kernel(abs_idx, table_flat)


def _mm_body(g1_ref, g2_ref, w1_ref, w2_ref, b_ref, o_ref):
    o_ref[...] = (
        jnp.dot(g1_ref[...], w1_ref[...], preferred_element_type=jnp.float32)
        + jnp.dot(g2_ref[...], w2_ref[...], preferred_element_type=jnp.float32)
        + b_ref[...]
    )


def kernel(input1, input2, embed_table, W, b):
    B = input1.shape[0]
    rows, D = embed_table.shape
    n_out = W.shape[0]

    tot = 2 * B
    per_w = tot // NW
    n_chunks = per_w // LANES
    pack = 128 // PITCH

    iii = jnp.concatenate([input1, input2]).astype(jnp.int32)
    abs_idx = iii.reshape(NW, n_chunks, 1, LANES) + (
        jnp.arange(D, dtype=jnp.int32) * rows
    ).reshape(1, 1, D, 1)
    table_flat = embed_table.T.reshape(-1)

    flat = _sc_gather(abs_idx, table_flat, D)
    g_view = flat.reshape(tot * PITCH // 128, 128)

    wp1 = jnp.pad(W[:, :D].T, ((0, PITCH - D), (0, 0)))
    wp2 = jnp.pad(W[:, D:].T, ((0, PITCH - D), (0, 0)))
    eye = jnp.eye(pack, dtype=jnp.float32)
    ww1 = jnp.kron(eye, wp1)
    ww2 = jnp.kron(eye, wp2)
    bias_v = jnp.tile(b, pack).reshape(1, pack * n_out)

    m_view = B * PITCH // 128
    n_wide = pack * n_out
    BM = 256
    grid = m_view // BM
    out_v = pl.pallas_call(
        _mm_body,
        grid=(grid,),
        in_specs=[
            pl.BlockSpec((BM, 128), lambda i: (i, 0)),
            pl.BlockSpec((BM, 128), lambda i, g=grid: (i + g, 0)),
            pl.BlockSpec((128, n_wide), lambda i: (0, 0)),
            pl.BlockSpec((128, n_wide), lambda i: (0, 0)),
            pl.BlockSpec((1, n_wide), lambda i: (0, 0)),
        ],
        out_specs=pl.BlockSpec((BM, n_wide), lambda i: (i, 0)),
        out_shape=jax.ShapeDtypeStruct((m_view, n_wide), jnp.float32),
    )(g_view, g_view, ww1, ww2, bias_v)
    return out_v.reshape(B, n_out)


# speedup vs baseline: 7.6211x; 7.6211x over previous
import functools

import jax
import jax.numpy as jnp
from jax import lax
from jax.experimental import pallas as pl
from jax.experimental.pallas import tpu as pltpu
from jax.experimental.pallas import tpu_sc as plsc

NC = 2
NS = 16
NW = NC * NS
LANES = 128
PITCH = 16


@functools.partial(jax.jit, static_argnums=(2,))
def _sc_gather(abs_idx, table_flat, d):
    n_chunks = abs_idx.shape[1]
    per_w = n_chunks * LANES
    flat_w = per_w * PITCH
    mesh = plsc.VectorSubcoreMesh(core_axis_name="c", subcore_axis_name="s")

    @functools.partial(
        pl.kernel,
        out_type=jax.ShapeDtypeStruct((NW * flat_w,), jnp.float32),
        mesh=mesh,
        scratch_types=[
            pltpu.VMEM((n_chunks, d, LANES), jnp.int32),
            pltpu.VMEM((d, per_w), jnp.float32),
            pltpu.VMEM((flat_w,), jnp.float32),
            pltpu.SemaphoreType.DMA,
            pltpu.SemaphoreType.DMA,
        ],
        compiler_params=pltpu.CompilerParams(needs_layout_passes=False),
    )
    def gather_kernel(idx_hbm, table_hbm, out_hbm, idx_v, feat_v, g_v, sem, sem2):
        wid = lax.axis_index("s") * NC + lax.axis_index("c")
        pltpu.sync_copy(idx_hbm.at[wid], idx_v)

        zeros16 = jnp.zeros((16,), jnp.float32)

        def zero_body(z, _):
            g_v[pl.ds(z * 16, 16)] = zeros16
            return 0

        lax.fori_loop(0, flat_w // 16, zero_body, 0)

        def chunk_body(c, _):
            copies = []
            for f in range(d):
                copies.append(
                    pltpu.async_copy(
                        table_hbm.at[idx_v.at[c, f]],
                        feat_v.at[f, pl.ds(c * LANES, LANES)],
                        sem,
                    )
                )
            for cp in copies:
                cp.wait()
            return 0

        lax.fori_loop(0, n_chunks, chunk_body, 0)

        lane16 = lax.iota(jnp.int32, 16) * PITCH

        def asm_f(f, _):
            def asm_g(g, _):
                v = feat_v[f, pl.ds(g * 16, 16)]
                plsc.store_scatter(g_v, [lane16 + (g * (16 * PITCH) + f)], v)
                return 0

            lax.fori_loop(0, per_w // 16, asm_g, 0)
            return 0

        lax.fori_loop(0, d, asm_f, 0)

        pltpu.async_copy(g_v, out_hbm.at[pl.ds(wid * flat_w, flat_w)], sem2).wait()

    return gather_kernel(abs_idx, table_flat)


def _flat_body(i_ref, o_ref):
    k = o_ref.shape[0] // 8
    x = i_ref[...].reshape(8, k, 128)
    o_ref[...] = jnp.swapaxes(x, 0, 1).reshape(k * 8, 128)


def _flatten_tiles(table_t, lane_tiles, kc):
    d_slabs = (table_t.shape[0] + 7) // 8
    n_c = lane_tiles // kc
    return pl.pallas_call(
        _flat_body,
        grid=(d_slabs, n_c),
        in_specs=[pl.BlockSpec((8, kc * 128), lambda r, c: (r, c))],
        out_specs=pl.BlockSpec(
            (kc * 8, 128), lambda r, c, n=n_c: (r * n + c, 0)
        ),
        out_shape=jax.ShapeDtypeStruct((d_slabs * lane_tiles * 8, 128), jnp.float32),
    )(table_t)


def _mm_body(g1_ref, g2_ref, w1_ref, w2_ref, b_ref, o_ref):
    o_ref[...] = (
        jnp.dot(g1_ref[...], w1_ref[...], preferred_element_type=jnp.float32)
        + jnp.dot(g2_ref[...], w2_ref[...], preferred_element_type=jnp.float32)
        + b_ref[...]
    )


def kernel(input1, input2, embed_table, W, b):
    B = input1.shape[0]
    rows, D = embed_table.shape
    n_out = W.shape[0]

    tot = 2 * B
    per_w = tot // NW
    n_chunks = per_w // LANES
    pack = 128 // PITCH

    lane_tiles = (rows + 127) // 128
    kc = 601

    iii = jnp.concatenate([input1, input2]).astype(jnp.int32)
    fr = jnp.arange(D, dtype=jnp.int32)
    slab_off = (fr // 8) * (lane_tiles * 1024) + (fr % 8) * 128
    abs_idx = ((iii >> 7) * 1024 + (iii & 127)).reshape(NW, n_chunks, 1, LANES) + (
        slab_off
    ).reshape(1, 1, D, 1)
    table_flat = _flatten_tiles(embed_table.T, lane_tiles, kc).reshape(-1)

    flat = _sc_gather(abs_idx, table_flat, D)
    g_view = flat.reshape(tot * PITCH // 128, 128)

    wp1 = jnp.pad(W[:, :D].T, ((0, PITCH - D), (0, 0)))
    wp2 = jnp.pad(W[:, D:].T, ((0, PITCH - D), (0, 0)))
    eye = jnp.eye(pack, dtype=jnp.float32)
    ww1 = jnp.kron(eye, wp1)
    ww2 = jnp.kron(eye, wp2)
    bias_v = jnp.tile(b, pack).reshape(1, pack * n_out)

    m_view = B * PITCH // 128
    n_wide = pack * n_out
    BM = 256
    grid = m_view // BM
    out_v = pl.pallas_call(
        _mm_body,
        grid=(grid,),
        in_specs=[
            pl.BlockSpec((BM, 128), lambda i: (i, 0)),
            pl.BlockSpec((BM, 128), lambda i, g=grid: (i + g, 0)),
            pl.BlockSpec((128, n_wide), lambda i: (0, 0)),
            pl.BlockSpec((128, n_wide), lambda i: (0, 0)),
            pl.BlockSpec((1, n_wide), lambda i: (0, 0)),
        ],
        out_specs=pl.BlockSpec((BM, n_wide), lambda i: (i, 0)),
        out_shape=jax.ShapeDtypeStruct((m_view, n_wide), jnp.float32),
    )(g_view, g_view, ww1, ww2, bias_v)
    return out_v.reshape(B, n_out)


# speedup vs baseline: 7.6602x; 1.0051x over previous
import functools

import jax
import jax.numpy as jnp
from jax import lax
from jax.experimental import pallas as pl
from jax.experimental.pallas import tpu as pltpu
from jax.experimental.pallas import tpu_sc as plsc

NC = 2
NS = 16
NW = NC * NS
LANES = 128
PITCH = 16


@functools.partial(jax.jit, static_argnums=(3,))
def _sc_gather(abs_idx, t0_flat, t1_flat, d):
    n_chunks = abs_idx.shape[1]
    per_w = n_chunks * LANES
    flat_w = per_w * PITCH
    mesh = plsc.VectorSubcoreMesh(core_axis_name="c", subcore_axis_name="s")

    @functools.partial(
        pl.kernel,
        out_type=jax.ShapeDtypeStruct((NW * flat_w,), jnp.float32),
        mesh=mesh,
        scratch_types=[
            pltpu.VMEM((n_chunks, d, LANES), jnp.int32),
            pltpu.VMEM((d, per_w), jnp.float32),
            pltpu.VMEM((flat_w,), jnp.float32),
            pltpu.SemaphoreType.DMA,
            pltpu.SemaphoreType.DMA,
        ],
        compiler_params=pltpu.CompilerParams(needs_layout_passes=False),
    )
    def gather_kernel(idx_hbm, t0_hbm, t1_hbm, out_hbm, idx_v, feat_v, g_v, sem, sem2):
        wid = lax.axis_index("s") * NC + lax.axis_index("c")
        pltpu.sync_copy(idx_hbm.at[wid], idx_v)

        zeros16 = jnp.zeros((16,), jnp.float32)

        def zero_body(z, _):
            g_v[pl.ds(z * 16, 16)] = zeros16
            return 0

        lax.fori_loop(0, flat_w // 16, zero_body, 0)

        def chunk_body(c, _):
            copies = []
            for f in range(d):
                src = t0_hbm if f < 8 else t1_hbm
                copies.append(
                    pltpu.async_copy(
                        src.at[idx_v.at[c, f]],
                        feat_v.at[f, pl.ds(c * LANES, LANES)],
                        sem,
                    )
                )
            for cp in copies:
                cp.wait()
            return 0

        lax.fori_loop(0, n_chunks, chunk_body, 0)

        lane16 = lax.iota(jnp.int32, 16) * PITCH

        def asm_f(f, _):
            def asm_g(g, _):
                v = feat_v[f, pl.ds(g * 16, 16)]
                plsc.store_scatter(g_v, [lane16 + (g * (16 * PITCH) + f)], v)
                return 0

            lax.fori_loop(0, per_w // 16, asm_g, 0)
            return 0

        lax.fori_loop(0, d, asm_f, 0)

        pltpu.async_copy(g_v, out_hbm.at[pl.ds(wid * flat_w, flat_w)], sem2).wait()

    return gather_kernel(abs_idx, t0_flat, t1_flat)


def _flat_body(i_ref, o_ref):
    rr = o_ref.shape[0] // (i_ref.shape[1] // 128)
    k = o_ref.shape[0] // rr
    x = i_ref[...].reshape(8, k, 128)
    o_ref[...] = jnp.swapaxes(x[:rr], 0, 1).reshape(k * rr, 128)


def _flatten_slab(table_t, slab, rr, lane_tiles, kc):
    n_c = -(-lane_tiles // kc)
    return pl.pallas_call(
        _flat_body,
        grid=(n_c,),
        in_specs=[pl.BlockSpec((8, kc * 128), lambda c, s=slab: (s, c))],
        out_specs=pl.BlockSpec((kc * rr, 128), lambda c: (c, 0)),
        out_shape=jax.ShapeDtypeStruct((n_c * kc * rr, 128), jnp.float32),
    )(table_t)


def _mm_body(g1_ref, g2_ref, w1_ref, w2_ref, b_ref, o_ref):
    o_ref[...] = (
        jnp.dot(g1_ref[...], w1_ref[...], preferred_element_type=jnp.float32)
        + jnp.dot(g2_ref[...], w2_ref[...], preferred_element_type=jnp.float32)
        + b_ref[...]
    )


def kernel(input1, input2, embed_table, W, b):
    B = input1.shape[0]
    rows, D = embed_table.shape
    n_out = W.shape[0]

    tot = 2 * B
    per_w = tot // NW
    n_chunks = per_w // LANES
    pack = 128 // PITCH

    lane_tiles = (rows + 127) // 128
    kc = 601

    d1 = D - 8

    iii = jnp.concatenate([input1, input2]).astype(jnp.int32)
    fr = jnp.arange(D, dtype=jnp.int32)
    f_off = jnp.where(fr < 8, fr * 128, (fr - 8) * 128)
    f_pitch = jnp.where(fr < 8, 1024, d1 * 128)
    abs_idx = (iii >> 7).reshape(NW, n_chunks, 1, LANES) * f_pitch.reshape(
        1, 1, D, 1
    ) + (iii & 127).reshape(NW, n_chunks, 1, LANES) + f_off.reshape(1, 1, D, 1)
    table_t = embed_table.T
    t0_flat = _flatten_slab(table_t, 0, 8, lane_tiles, kc).reshape(-1)
    t1_flat = _flatten_slab(table_t, 1, d1, lane_tiles, 604).reshape(-1)

    flat = _sc_gather(abs_idx, t0_flat, t1_flat, D)
    g2d = flat.reshape(tot, PITCH)

    wp1 = jnp.pad(W[:, :D].T, ((0, PITCH - D), (0, 0)))
    wp2 = jnp.pad(W[:, D:].T, ((0, PITCH - D), (0, 0)))
    bias_v = b.reshape(1, n_out)

    BM = 2048
    grid = B // BM
    out = pl.pallas_call(
        _mm_body,
        grid=(grid,),
        in_specs=[
            pl.BlockSpec((BM, PITCH), lambda i: (i, 0)),
            pl.BlockSpec((BM, PITCH), lambda i, g=grid: (i + g, 0)),
            pl.BlockSpec((PITCH, n_out), lambda i: (0, 0)),
            pl.BlockSpec((PITCH, n_out), lambda i: (0, 0)),
            pl.BlockSpec((1, n_out), lambda i: (0, 0)),
        ],
        out_specs=pl.BlockSpec((BM, n_out), lambda i: (i, 0)),
        out_shape=jax.ShapeDtypeStruct((B, n_out), jnp.float32),
    )(g2d, g2d, wp1, wp2, bias_v)
    return out


# speedup vs baseline: 8.6851x; 1.1338x over previous
import functools

import jax
import jax.numpy as jnp
from jax import lax
from jax.experimental import pallas as pl
from jax.experimental.pallas import tpu as pltpu
from jax.experimental.pallas import tpu_sc as plsc

NC = 2
NS = 16
NW = NC * NS
LANES = 128
PITCH = 16


@functools.partial(jax.jit, static_argnums=(3,))
def _sc_gather(abs_idx, t0_flat, t1_flat, d):
    n_chunks = abs_idx.shape[1]
    per_w = n_chunks * LANES
    flat_w = per_w * PITCH
    mesh = plsc.VectorSubcoreMesh(core_axis_name="c", subcore_axis_name="s")

    @functools.partial(
        pl.kernel,
        out_type=jax.ShapeDtypeStruct((NW * flat_w,), jnp.float32),
        mesh=mesh,
        scratch_types=[
            pltpu.VMEM((n_chunks, d, LANES), jnp.int32),
            pltpu.VMEM((d, per_w), jnp.float32),
            pltpu.VMEM((flat_w,), jnp.float32),
            pltpu.SemaphoreType.DMA,
            pltpu.SemaphoreType.DMA,
        ],
        compiler_params=pltpu.CompilerParams(needs_layout_passes=False),
    )
    def gather_kernel(idx_hbm, t0_hbm, t1_hbm, out_hbm, idx_v, feat_v, g_v, sem, sem2):
        wid = lax.axis_index("s") * NC + lax.axis_index("c")
        pltpu.sync_copy(idx_hbm.at[wid], idx_v)

        zeros16 = jnp.zeros((16,), jnp.float32)

        def zero_body(z, _):
            g_v[pl.ds(z * 16, 16)] = zeros16
            return 0

        lax.fori_loop(0, flat_w // 16, zero_body, 0)

        def chunk_body(c, _):
            copies = []
            for f in range(d):
                src = t0_hbm if f < 8 else t1_hbm
                copies.append(
                    pltpu.async_copy(
                        src.at[idx_v.at[c, f]],
                        feat_v.at[f, pl.ds(c * LANES, LANES)],
                        sem,
                    )
                )
            for cp in copies:
                cp.wait()
            return 0

        lax.fori_loop(0, n_chunks, chunk_body, 0)

        lane16 = lax.iota(jnp.int32, 16) * PITCH

        def asm_f(f, _):
            def asm_g(g, _):
                v = feat_v[f, pl.ds(g * 16, 16)]
                plsc.store_scatter(g_v, [lane16 + (g * (16 * PITCH) + f)], v)
                return 0

            lax.fori_loop(0, per_w // 16, asm_g, 0)
            return 0

        lax.fori_loop(0, d, asm_f, 0)

        pltpu.async_copy(g_v, out_hbm.at[pl.ds(wid * flat_w, flat_w)], sem2).wait()

    return gather_kernel(abs_idx, t0_flat, t1_flat)


def _flat_body(i_ref, o_ref):
    rr = o_ref.shape[0] // (i_ref.shape[1] // 128)
    k = o_ref.shape[0] // rr
    x = i_ref[...].reshape(8, k, 128)
    o_ref[...] = jnp.swapaxes(x[:rr], 0, 1).reshape(k * rr, 128)


def _flatten_slab(table_t, slab, rr, lane_tiles, kc):
    n_c = -(-lane_tiles // kc)
    return pl.pallas_call(
        _flat_body,
        grid=(n_c,),
        in_specs=[pl.BlockSpec((8, kc * 128), lambda c, s=slab: (s, c))],
        out_specs=pl.BlockSpec((kc * rr, 128), lambda c: (c, 0)),
        out_shape=jax.ShapeDtypeStruct((n_c * kc * rr, 128), jnp.float32),
    )(table_t)


def _mm_body(g1_ref, g2_ref, w1_ref, w2_ref, b_ref, o_ref):
    acc = (
        jnp.dot(g1_ref[...], w1_ref[...], preferred_element_type=jnp.float32)
        + jnp.dot(g2_ref[...], w2_ref[...], preferred_element_type=jnp.float32)
        + b_ref[...]
    )
    o_ref[...] = acc.reshape(o_ref.shape)


def kernel(input1, input2, embed_table, W, b):
    B = input1.shape[0]
    rows, D = embed_table.shape
    n_out = W.shape[0]

    tot = 2 * B
    per_w = tot // NW
    n_chunks = per_w // LANES
    pack = 128 // PITCH

    lane_tiles = (rows + 127) // 128
    kc = 601

    d1 = D - 8

    iii = jnp.concatenate([input1, input2]).astype(jnp.int32)
    fr = jnp.arange(D, dtype=jnp.int32)
    f_off = jnp.where(fr < 8, fr * 128, (fr - 8) * 128)
    f_pitch = jnp.where(fr < 8, 1024, d1 * 128)
    abs_idx = (iii >> 7).reshape(NW, n_chunks, 1, LANES) * f_pitch.reshape(
        1, 1, D, 1
    ) + (iii & 127).reshape(NW, n_chunks, 1, LANES) + f_off.reshape(1, 1, D, 1)
    table_t = embed_table.T
    t0_flat = _flatten_slab(table_t, 0, 8, lane_tiles, kc).reshape(-1)
    t1_flat = _flatten_slab(table_t, 1, d1, lane_tiles, 604).reshape(-1)

    flat = _sc_gather(abs_idx, t0_flat, t1_flat, D)
    g_view = flat.reshape(tot * PITCH // 128, 128)

    wp1 = jnp.pad(W[:, :D].T, ((0, PITCH - D), (0, 0)))
    wp2 = jnp.pad(W[:, D:].T, ((0, PITCH - D), (0, 0)))
    eye = jnp.eye(pack, dtype=jnp.float32)
    ww1 = jnp.kron(eye, wp1)
    ww2 = jnp.kron(eye, wp2)
    bias_v = jnp.tile(b, pack).reshape(1, pack * n_out)

    m_view = B * PITCH // 128
    n_wide = pack * n_out
    BM = 256
    grid = m_view // BM
    out = pl.pallas_call(
        _mm_body,
        grid=(grid,),
        in_specs=[
            pl.BlockSpec((BM, 128), lambda i: (i, 0)),
            pl.BlockSpec((BM, 128), lambda i, g=grid: (i + g, 0)),
            pl.BlockSpec((128, n_wide), lambda i: (0, 0)),
            pl.BlockSpec((128, n_wide), lambda i: (0, 0)),
            pl.BlockSpec((1, n_wide), lambda i: (0, 0)),
        ],
        out_specs=pl.BlockSpec((BM * pack, n_out), lambda i: (i, 0)),
        out_shape=jax.ShapeDtypeStruct((B, n_out), jnp.float32),
    )(g_view, g_view, ww1, ww2, bias_v)
    return out


# speedup vs baseline: 9.6337x; 1.1092x over previous
import functools

import jax
import jax.numpy as jnp
from jax import lax
from jax.experimental import pallas as pl
from jax.experimental.pallas import tpu as pltpu
from jax.experimental.pallas import tpu_sc as plsc

NC = 2
NS = 16
NW = NC * NS
LANES = 128
PITCH = 16


@functools.partial(jax.jit, static_argnums=(3,))
def _sc_gather(abs_idx, t0_flat, t1_flat, d):
    n_chunks = abs_idx.shape[1]
    per_w = n_chunks * LANES
    flat_w = per_w * PITCH
    mesh = plsc.VectorSubcoreMesh(core_axis_name="c", subcore_axis_name="s")

    @functools.partial(
        pl.kernel,
        out_type=jax.ShapeDtypeStruct((NW * flat_w,), jnp.float32),
        mesh=mesh,
        scratch_types=[
            pltpu.VMEM((n_chunks, d, LANES), jnp.int32),
            pltpu.VMEM((d * per_w,), jnp.float32),
            pltpu.VMEM((flat_w,), jnp.float32),
            pltpu.SemaphoreType.DMA,
            pltpu.SemaphoreType.DMA,
        ],
        compiler_params=pltpu.CompilerParams(needs_layout_passes=False),
    )
    def gather_kernel(idx_hbm, t0_hbm, t1_hbm, out_hbm, idx_v, feat_v, g_v, sem, sem2):
        wid = lax.axis_index("s") * NC + lax.axis_index("c")
        pltpu.sync_copy(idx_hbm.at[wid], idx_v)

        def chunk_body(c, _):
            for f in range(d):
                src = t0_hbm if f < 8 else t1_hbm
                pltpu.async_copy(
                    src.at[idx_v.at[c, f]],
                    feat_v.at[pl.ds((f * n_chunks + c) * LANES, LANES)],
                    sem,
                )
            return 0

        lax.fori_loop(0, n_chunks, chunk_body, 0)

        zeros16 = jnp.zeros((16,), jnp.float32)

        def zero_body(z, _):
            g_v[pl.ds(z * 16, 16)] = zeros16
            return 0

        lax.fori_loop(0, flat_w // 16, zero_body, 0)

        pltpu.make_async_copy(
            out_hbm.at[pl.ds(0, d * per_w)], feat_v, sem
        ).wait()

        lane16 = lax.iota(jnp.int32, 16) * PITCH

        def asm_f(f, _):
            def asm_g(g, _):
                v = feat_v[pl.ds(f * per_w + g * 16, 16)]
                plsc.store_scatter(g_v, [lane16 + (g * (16 * PITCH) + f)], v)
                return 0

            lax.fori_loop(0, per_w // 16, asm_g, 0)
            return 0

        lax.fori_loop(0, d, asm_f, 0)

        pltpu.async_copy(g_v, out_hbm.at[pl.ds(wid * flat_w, flat_w)], sem2).wait()

    return gather_kernel(abs_idx, t0_flat, t1_flat)


def _flat_body(i_ref, o_ref):
    rr = o_ref.shape[0] // (i_ref.shape[1] // 128)
    k = o_ref.shape[0] // rr
    x = i_ref[...].reshape(8, k, 128)
    o_ref[...] = jnp.swapaxes(x[:rr], 0, 1).reshape(k * rr, 128)


def _flatten_slab(table_t, slab, rr, lane_tiles, kc):
    n_c = -(-lane_tiles // kc)
    return pl.pallas_call(
        _flat_body,
        grid=(n_c,),
        in_specs=[pl.BlockSpec((8, kc * 128), lambda c, s=slab: (s, c))],
        out_specs=pl.BlockSpec((kc * rr, 128), lambda c: (c, 0)),
        out_shape=jax.ShapeDtypeStruct((n_c * kc * rr, 128), jnp.float32),
    )(table_t)


def _mm_body(g1_ref, g2_ref, w1_ref, w2_ref, b_ref, o_ref):
    acc = (
        jnp.dot(g1_ref[...], w1_ref[...], preferred_element_type=jnp.float32)
        + jnp.dot(g2_ref[...], w2_ref[...], preferred_element_type=jnp.float32)
        + b_ref[...]
    )
    o_ref[...] = acc.reshape(o_ref.shape)


def kernel(input1, input2, embed_table, W, b):
    B = input1.shape[0]
    rows, D = embed_table.shape
    n_out = W.shape[0]

    tot = 2 * B
    per_w = tot // NW
    n_chunks = per_w // LANES
    pack = 128 // PITCH

    lane_tiles = (rows + 127) // 128
    kc = 601

    d1 = D - 8

    iii = jnp.concatenate([input1, input2]).astype(jnp.int32)
    fr = jnp.arange(D, dtype=jnp.int32)
    f_off = jnp.where(fr < 8, fr * 128, (fr - 8) * 128)
    f_pitch = jnp.where(fr < 8, 1024, d1 * 128)
    abs_idx = (iii >> 7).reshape(NW, n_chunks, 1, LANES) * f_pitch.reshape(
        1, 1, D, 1
    ) + (iii & 127).reshape(NW, n_chunks, 1, LANES) + f_off.reshape(1, 1, D, 1)
    table_t = embed_table.T
    t0_flat = _flatten_slab(table_t, 0, 8, lane_tiles, kc).reshape(-1)
    t1_flat = _flatten_slab(table_t, 1, d1, lane_tiles, 604).reshape(-1)

    flat = _sc_gather(abs_idx, t0_flat, t1_flat, D)
    g_view = flat.reshape(tot * PITCH // 128, 128)

    wp1 = jnp.pad(W[:, :D].T, ((0, PITCH - D), (0, 0)))
    wp2 = jnp.pad(W[:, D:].T, ((0, PITCH - D), (0, 0)))
    eye = jnp.eye(pack, dtype=jnp.float32)
    ww1 = jnp.kron(eye, wp1)
    ww2 = jnp.kron(eye, wp2)
    bias_v = jnp.tile(b, pack).reshape(1, pack * n_out)

    m_view = B * PITCH // 128
    n_wide = pack * n_out
    BM = 256
    grid = m_view // BM
    out = pl.pallas_call(
        _mm_body,
        grid=(grid,),
        in_specs=[
            pl.BlockSpec((BM, 128), lambda i: (i, 0)),
            pl.BlockSpec((BM, 128), lambda i, g=grid: (i + g, 0)),
            pl.BlockSpec((128, n_wide), lambda i: (0, 0)),
            pl.BlockSpec((128, n_wide), lambda i: (0, 0)),
            pl.BlockSpec((1, n_wide), lambda i: (0, 0)),
        ],
        out_specs=pl.BlockSpec((BM * pack, n_out), lambda i: (i, 0)),
        out_shape=jax.ShapeDtypeStruct((B, n_out), jnp.float32),
    )(g_view, g_view, ww1, ww2, bias_v)
    return out


# speedup vs baseline: 11.1414x; 1.1565x over previous
import functools

import jax
import jax.numpy as jnp
from jax import lax
from jax.experimental import pallas as pl
from jax.experimental.pallas import tpu as pltpu
from jax.experimental.pallas import tpu_sc as plsc

NC = 2
NS = 16
NW = NC * NS
LANES = 128
PITCH = 16


@functools.partial(jax.jit, static_argnums=(3,))
def _sc_gather(abs_idx, t0_flat, t1_flat, d):
    n_chunks = abs_idx.shape[1]
    per_w = n_chunks * LANES
    flat_w = per_w * PITCH
    mesh = plsc.VectorSubcoreMesh(core_axis_name="c", subcore_axis_name="s")

    @functools.partial(
        pl.kernel,
        out_type=jax.ShapeDtypeStruct((NW * flat_w,), jnp.float32),
        mesh=mesh,
        scratch_types=[
            pltpu.VMEM((n_chunks, d, LANES), jnp.int32),
            pltpu.VMEM((d * per_w,), jnp.float32),
            pltpu.VMEM((flat_w,), jnp.float32),
            pltpu.SemaphoreType.DMA,
            pltpu.SemaphoreType.DMA,
        ],
        compiler_params=pltpu.CompilerParams(needs_layout_passes=False),
    )
    def gather_kernel(idx_hbm, t0_hbm, t1_hbm, out_hbm, idx_v, feat_v, g_v, sem, sem2):
        wid = lax.axis_index("s") * NC + lax.axis_index("c")
        pltpu.sync_copy(idx_hbm.at[wid], idx_v)

        def chunk_body(c, _):
            for f in range(d):
                src = t0_hbm if f < 8 else t1_hbm
                pltpu.async_copy(
                    src.at[idx_v.at[c, f]],
                    feat_v.at[pl.ds((f * n_chunks + c) * LANES, LANES)],
                    sem,
                )
            return 0

        lax.fori_loop(0, n_chunks, chunk_body, 0)

        zeros16 = jnp.zeros((16,), jnp.float32)

        def zero_body(z, _):
            g_v[pl.ds(z * 16, 16)] = zeros16
            return 0

        lax.fori_loop(0, flat_w // 16, zero_body, 0)

        pltpu.make_async_copy(
            out_hbm.at[pl.ds(0, d * per_w)], feat_v, sem
        ).wait()

        lane16 = lax.iota(jnp.int32, 16) * PITCH

        def asm_f(f, _):
            def asm_g(g, _):
                v = feat_v[pl.ds(f * per_w + g * 16, 16)]
                plsc.store_scatter(g_v, [lane16 + (g * (16 * PITCH) + f)], v)
                return 0

            lax.fori_loop(0, per_w // 16, asm_g, 0)
            return 0

        lax.fori_loop(0, d, asm_f, 0)

        pltpu.async_copy(g_v, out_hbm.at[pl.ds(wid * flat_w, flat_w)], sem2).wait()

    return gather_kernel(abs_idx, t0_flat, t1_flat)


def _sc_flatten_slab1(table_t, tail_src, lane_tiles, d1):
    kb = 49
    nb = 5
    per = kb * nb
    last_full = (table_t.shape[1] // 128) - 1 + 1
    tail_q0 = 7791
    tail_w = 21 * 128
    tail_k = 21
    out_len = (lane_tiles + 7) // 8 * 8 * d1 * 128
    mesh = plsc.VectorSubcoreMesh(core_axis_name="c", subcore_axis_name="s")

    @functools.partial(
        pl.kernel,
        out_type=jax.ShapeDtypeStruct((out_len,), jnp.float32),
        mesh=mesh,
        scratch_types=[
            pltpu.VMEM((d1, kb * 128), jnp.float32),
            pltpu.VMEM((kb * d1 * 128,), jnp.float32),
            pltpu.SemaphoreType.DMA,
        ],
        compiler_params=pltpu.CompilerParams(needs_layout_passes=False),
    )
    def slab1_kernel(table_hbm, tail_hbm, out_hbm, buf, out_v, sem):
        wid = lax.axis_index("s") * NC + lax.axis_index("c")
        q0w = wid * per

        def fill(n_tiles):
            def fill_t(t, _):
                for r in range(d1):
                    for s in range(8):
                        out_v[pl.ds((t * d1 + r) * 128 + s * 16, 16)] = buf[
                            r, pl.ds(t * 128 + s * 16, 16)
                        ]
                return 0

            lax.fori_loop(0, n_tiles, fill_t, 0)

        def batch_body(bi, _):
            qb = q0w + bi * kb

            @pl.when(qb + kb <= last_full)
            def _():
                pltpu.sync_copy(
                    table_hbm.at[pl.ds(8, d1), pl.ds(qb * 128, kb * 128)], buf
                )
                fill(kb)
                pltpu.async_copy(
                    out_v, out_hbm.at[pl.ds(qb * d1 * 128, kb * d1 * 128)], sem
                ).wait()

            return 0

        lax.fori_loop(0, nb, batch_body, 0)

        @pl.when(wid == NW - 1)
        def _():
            pltpu.sync_copy(
                table_hbm.at[pl.ds(8, d1), pl.ds(tail_q0 * 128, tail_w)],
                buf.at[pl.ds(0, d1), pl.ds(0, tail_w)],
            )
            fill(tail_k)
            pltpu.async_copy(
                out_v.at[pl.ds(0, tail_k * d1 * 128)],
                out_hbm.at[pl.ds(tail_q0 * d1 * 128, tail_k * d1 * 128)],
                sem,
            ).wait()
            pltpu.sync_copy(tail_hbm, out_v.at[pl.ds(0, d1 * 128)])
            pltpu.async_copy(
                out_v.at[pl.ds(0, d1 * 128)],
                out_hbm.at[pl.ds((tail_q0 + tail_k) * d1 * 128, d1 * 128)],
                sem,
            ).wait()

    return slab1_kernel(table_t, tail_src)


def _flat_body(i_ref, o_ref):
    rr = o_ref.shape[0] // (i_ref.shape[1] // 128)
    k = o_ref.shape[0] // rr
    x = i_ref[...].reshape(8, k, 128)
    o_ref[...] = jnp.swapaxes(x[:rr], 0, 1).reshape(k * rr, 128)


def _flatten_slab(table_t, slab, rr, lane_tiles, kc):
    n_c = -(-lane_tiles // kc)
    return pl.pallas_call(
        _flat_body,
        grid=(n_c,),
        in_specs=[pl.BlockSpec((8, kc * 128), lambda c, s=slab: (s, c))],
        out_specs=pl.BlockSpec((kc * rr, 128), lambda c: (c, 0)),
        out_shape=jax.ShapeDtypeStruct((n_c * kc * rr, 128), jnp.float32),
    )(table_t)


def _mm_body(g1_ref, g2_ref, w1_ref, w2_ref, b_ref, o_ref):
    acc = (
        jnp.dot(g1_ref[...], w1_ref[...], preferred_element_type=jnp.float32)
        + jnp.dot(g2_ref[...], w2_ref[...], preferred_element_type=jnp.float32)
        + b_ref[...]
    )
    o_ref[...] = acc.reshape(o_ref.shape)


def kernel(input1, input2, embed_table, W, b):
    B = input1.shape[0]
    rows, D = embed_table.shape
    n_out = W.shape[0]

    tot = 2 * B
    per_w = tot // NW
    n_chunks = per_w // LANES
    pack = 128 // PITCH

    lane_tiles = (rows + 127) // 128
    kc = 601

    d1 = D - 8

    iii = jnp.concatenate([input1, input2]).astype(jnp.int32)
    fr = jnp.arange(D, dtype=jnp.int32)
    f_off = jnp.where(fr < 8, fr * 128, (fr - 8) * 128)
    f_pitch = jnp.where(fr < 8, 1024, d1 * 128)
    abs_idx = (iii >> 7).reshape(NW, n_chunks, 1, LANES) * f_pitch.reshape(
        1, 1, D, 1
    ) + (iii & 127).reshape(NW, n_chunks, 1, LANES) + f_off.reshape(1, 1, D, 1)
    table_t = embed_table.T
    t0_flat = _flatten_slab(table_t, 0, 8, lane_tiles, kc).reshape(-1)
    tail_src = jnp.pad(
        table_t[8:, (lane_tiles - 1) * 128 :], ((0, 0), (0, lane_tiles * 128 - rows))
    ).reshape(-1)
    t1_flat = _sc_flatten_slab1(table_t, tail_src, lane_tiles, d1)

    flat = _sc_gather(abs_idx, t0_flat, t1_flat, D)
    g_view = flat.reshape(tot * PITCH // 128, 128)

    wp1 = jnp.pad(W[:, :D].T, ((0, PITCH - D), (0, 0)))
    wp2 = jnp.pad(W[:, D:].T, ((0, PITCH - D), (0, 0)))
    eye = jnp.eye(pack, dtype=jnp.float32)
    ww1 = jnp.kron(eye, wp1)
    ww2 = jnp.kron(eye, wp2)
    bias_v = jnp.tile(b, pack).reshape(1, pack * n_out)

    m_view = B * PITCH // 128
    n_wide = pack * n_out
    BM = 256
    grid = m_view // BM
    out = pl.pallas_call(
        _mm_body,
        grid=(grid,),
        in_specs=[
            pl.BlockSpec((BM, 128), lambda i: (i, 0)),
            pl.BlockSpec((BM, 128), lambda i, g=grid: (i + g, 0)),
            pl.BlockSpec((128, n_wide), lambda i: (0, 0)),
            pl.BlockSpec((128, n_wide), lambda i: (0, 0)),
            pl.BlockSpec((1, n_wide), lambda i: (0, 0)),
        ],
        out_specs=pl.BlockSpec((BM * pack, n_out), lambda i: (i, 0)),
        out_shape=jax.ShapeDtypeStruct((B, n_out), jnp.float32),
    )(g_view, g_view, ww1, ww2, bias_v)
    return out


# speedup vs baseline: 11.4141x; 1.0245x over previous
import functools

import jax
import jax.numpy as jnp
from jax import lax
from jax.experimental import pallas as pl
from jax.experimental.pallas import tpu as pltpu
from jax.experimental.pallas import tpu_sc as plsc

NC = 2
NS = 16
NW = NC * NS
LANES = 128
PITCH = 16


@functools.partial(jax.jit, static_argnums=(3,))
def _sc_gather(iii2d, t0_flat, t1_flat, d):
    per_w = iii2d.shape[1]
    n_chunks = per_w // LANES
    flat_w = per_w * PITCH
    d1 = d - 8
    mesh = plsc.VectorSubcoreMesh(core_axis_name="c", subcore_axis_name="s")

    @functools.partial(
        pl.kernel,
        out_type=jax.ShapeDtypeStruct((NW * flat_w,), jnp.float32),
        mesh=mesh,
        scratch_types=[
            pltpu.VMEM((per_w,), jnp.int32),
            pltpu.VMEM((d * per_w,), jnp.int32),
            pltpu.VMEM((d * per_w,), jnp.float32),
            pltpu.VMEM((flat_w,), jnp.float32),
            pltpu.SemaphoreType.DMA((d,)),
            pltpu.SemaphoreType.DMA,
        ],
        compiler_params=pltpu.CompilerParams(needs_layout_passes=False),
    )
    def gather_kernel(
        iii_hbm, t0_hbm, t1_hbm, out_hbm, idxr_v, idx_v, feat_v, g_v, sems, sem2
    ):
        wid = lax.axis_index("s") * NC + lax.axis_index("c")
        pltpu.sync_copy(iii_hbm.at[wid], idxr_v)

        def addr_body(v, _):
            x = idxr_v[pl.ds(v * 16, 16)]
            q = lax.shift_right_logical(x, 7)
            r = jnp.bitwise_and(x, 127)
            b0 = q * 1024 + r
            b1 = q * (d1 * 128) + r
            for f in range(d):
                base = b0 + f * 128 if f < 8 else b1 + (f - 8) * 128
                idx_v[pl.ds(f * per_w + v * 16, 16)] = base
            return 0

        lax.fori_loop(0, per_w // 16, addr_body, 0)

        def chunk_body(c, _):
            for f in range(d):
                src = t0_hbm if f < 8 else t1_hbm
                pltpu.async_copy(
                    src.at[idx_v.at[pl.ds((f * n_chunks + c) * LANES, LANES)]],
                    feat_v.at[pl.ds((f * n_chunks + c) * LANES, LANES)],
                    sems.at[f],
                )
            return 0

        lax.fori_loop(0, n_chunks, chunk_body, 0)

        zeros16 = jnp.zeros((16,), jnp.float32)

        def zero_body(z, _):
            g_v[pl.ds(z * 16, 16)] = zeros16
            return 0

        lax.fori_loop(0, flat_w // 16, zero_body, 0)

        lane16 = lax.iota(jnp.int32, 16) * PITCH

        for f in range(d):
            pltpu.make_async_copy(
                out_hbm.at[pl.ds(0, per_w)],
                feat_v.at[pl.ds(f * per_w, per_w)],
                sems.at[f],
            ).wait()

            def asm_g(g, _, f=f):
                v = feat_v[pl.ds(f * per_w + g * 16, 16)]
                plsc.store_scatter(g_v, [lane16 + (g * (16 * PITCH) + f)], v)
                return 0

            lax.fori_loop(0, per_w // 16, asm_g, 0)

        pltpu.async_copy(g_v, out_hbm.at[pl.ds(wid * flat_w, flat_w)], sem2).wait()

    return gather_kernel(iii2d, t0_flat, t1_flat)


def _sc_flatten_slab1(table_t, tail_src, lane_tiles, d1):
    kb = 49
    nb = 5
    per = kb * nb
    last_full = (table_t.shape[1] // 128) - 1 + 1
    tail_q0 = 7791
    tail_w = 21 * 128
    tail_k = 21
    out_len = (lane_tiles + 7) // 8 * 8 * d1 * 128
    mesh = plsc.VectorSubcoreMesh(core_axis_name="c", subcore_axis_name="s")

    @functools.partial(
        pl.kernel,
        out_type=jax.ShapeDtypeStruct((out_len,), jnp.float32),
        mesh=mesh,
        scratch_types=[
            pltpu.VMEM((d1, kb * 128), jnp.float32),
            pltpu.VMEM((kb * d1 * 128,), jnp.float32),
            pltpu.SemaphoreType.DMA,
        ],
        compiler_params=pltpu.CompilerParams(needs_layout_passes=False),
    )
    def slab1_kernel(table_hbm, tail_hbm, out_hbm, buf, out_v, sem):
        wid = lax.axis_index("s") * NC + lax.axis_index("c")
        q0w = wid * per

        def fill(n_tiles):
            def fill_t(t, _):
                for r in range(d1):
                    for s in range(8):
                        out_v[pl.ds((t * d1 + r) * 128 + s * 16, 16)] = buf[
                            r, pl.ds(t * 128 + s * 16, 16)
                        ]
                return 0

            lax.fori_loop(0, n_tiles, fill_t, 0)

        def batch_body(bi, _):
            qb = q0w + bi * kb

            @pl.when(qb + kb <= last_full)
            def _():
                pltpu.sync_copy(
                    table_hbm.at[pl.ds(8, d1), pl.ds(qb * 128, kb * 128)], buf
                )
                fill(kb)
                pltpu.async_copy(
                    out_v, out_hbm.at[pl.ds(qb * d1 * 128, kb * d1 * 128)], sem
                ).wait()

            return 0

        lax.fori_loop(0, nb, batch_body, 0)

        @pl.when(wid == NW - 1)
        def _():
            pltpu.sync_copy(
                table_hbm.at[pl.ds(8, d1), pl.ds(tail_q0 * 128, tail_w)],
                buf.at[pl.ds(0, d1), pl.ds(0, tail_w)],
            )
            fill(tail_k)
            pltpu.async_copy(
                out_v.at[pl.ds(0, tail_k * d1 * 128)],
                out_hbm.at[pl.ds(tail_q0 * d1 * 128, tail_k * d1 * 128)],
                sem,
            ).wait()
            pltpu.sync_copy(tail_hbm, out_v.at[pl.ds(0, d1 * 128)])
            pltpu.async_copy(
                out_v.at[pl.ds(0, d1 * 128)],
                out_hbm.at[pl.ds((tail_q0 + tail_k) * d1 * 128, d1 * 128)],
                sem,
            ).wait()

    return slab1_kernel(table_t, tail_src)


def _flat_body(i_ref, o_ref):
    rr = o_ref.shape[0] // (i_ref.shape[1] // 128)
    k = o_ref.shape[0] // rr
    x = i_ref[...].reshape(8, k, 128)
    o_ref[...] = jnp.swapaxes(x[:rr], 0, 1).reshape(k * rr, 128)


def _flatten_slab(table_t, slab, rr, lane_tiles, kc):
    n_c = -(-lane_tiles // kc)
    return pl.pallas_call(
        _flat_body,
        grid=(n_c,),
        in_specs=[pl.BlockSpec((8, kc * 128), lambda c, s=slab: (s, c))],
        out_specs=pl.BlockSpec((kc * rr, 128), lambda c: (c, 0)),
        out_shape=jax.ShapeDtypeStruct((n_c * kc * rr, 128), jnp.float32),
    )(table_t)


def _mm_body(g1_ref, g2_ref, w1_ref, w2_ref, b_ref, o_ref):
    acc = jnp.dot(
        g1_ref[...], w1_ref[...], preferred_element_type=jnp.float32
    ) + jnp.dot(g2_ref[...], w2_ref[...], preferred_element_type=jnp.float32)
    o_ref[...] = acc.reshape(o_ref.shape) + b_ref[...]


def kernel(input1, input2, embed_table, W, b):
    B = input1.shape[0]
    rows, D = embed_table.shape
    n_out = W.shape[0]

    tot = 2 * B
    per_w = tot // NW
    n_chunks = per_w // LANES
    pack = 128 // PITCH

    lane_tiles = (rows + 127) // 128
    kc = 601

    d1 = D - 8

    iii2d = jnp.concatenate([input1, input2]).astype(jnp.int32).reshape(NW, per_w)
    table_t = embed_table.T
    t0_flat = _flatten_slab(table_t, 0, 8, lane_tiles, kc).reshape(-1)
    tail_src = jnp.pad(
        table_t[8:, (lane_tiles - 1) * 128 :], ((0, 0), (0, lane_tiles * 128 - rows))
    ).reshape(-1)
    t1_flat = _sc_flatten_slab1(table_t, tail_src, lane_tiles, d1)

    flat = _sc_gather(iii2d, t0_flat, t1_flat, D)
    g_view = flat.reshape(tot * PITCH // 128, 128)

    wp1 = jnp.pad(W[:, :D].T, ((0, PITCH - D), (0, 0)))
    wp2 = jnp.pad(W[:, D:].T, ((0, PITCH - D), (0, 0)))
    eye = jnp.eye(pack, dtype=jnp.float32)
    ww1 = jnp.kron(eye, wp1)
    ww2 = jnp.kron(eye, wp2)
    bias_v = b.reshape(1, n_out)

    m_view = B * PITCH // 128
    n_wide = pack * n_out
    BM = 256
    grid = m_view // BM
    out = pl.pallas_call(
        _mm_body,
        grid=(grid,),
        in_specs=[
            pl.BlockSpec((BM, 128), lambda i: (i, 0)),
            pl.BlockSpec((BM, 128), lambda i, g=grid: (i + g, 0)),
            pl.BlockSpec((128, n_wide), lambda i: (0, 0)),
            pl.BlockSpec((128, n_wide), lambda i: (0, 0)),
            pl.BlockSpec((1, n_out), lambda i: (0, 0)),
        ],
        out_specs=pl.BlockSpec((BM * pack, n_out), lambda i: (i, 0)),
        out_shape=jax.ShapeDtypeStruct((B, n_out), jnp.float32),
    )(g_view, g_view, ww1, ww2, bias_v)
    return out
